# Initial kernel scaffold; baseline (speedup 1.0000x reference)
import jax
import jax.numpy as jnp
from jax.experimental import pallas as pl


def kernel(feats, edge_index, edge_attr, Wq, bq, Wk, bk, Wv, bv, Wo, bo, We, be, W1, b1, W2, b2, ln_g, ln_b):
    raise NotImplementedError("write your pallas kernel here")


# speedup vs baseline: 2.8558x; 2.8558x over previous
import functools
import math

import jax
import jax.numpy as jnp
from jax.experimental import pallas as pl
from jax.experimental.pallas import tpu as pltpu

N = 10000
E = 320000
EMBED = 128
NHEAD = 8
HDIM = EMBED // NHEAD
DHID = 4 * EMBED


def _ln(x, g, b):
    m = jnp.mean(x, axis=-1, keepdims=True)
    v = jnp.mean((x - m) ** 2, axis=-1, keepdims=True)
    return (x - m) * jax.lax.rsqrt(v + 1e-5) * g + b


def _gelu(x):
    return 0.5 * x * (1.0 + jax.lax.erf(x * (1.0 / math.sqrt(2.0))))


def _qkv_body(f_ref, w_ref, b_ref, o_ref):
    o_ref[...] = (
        jnp.dot(f_ref[...], w_ref[...], preferred_element_type=jnp.float32)
        + b_ref[...]
    )


def _qkv(feats, Wqkv_t, bqkv):
    BN = 1000
    return pl.pallas_call(
        _qkv_body,
        grid=(N // BN,),
        in_specs=[
            pl.BlockSpec((BN, EMBED), lambda i: (i, 0)),
            pl.BlockSpec((EMBED, 3 * EMBED), lambda i: (0, 0)),
            pl.BlockSpec((1, 3 * EMBED), lambda i: (0, 0)),
        ],
        out_specs=pl.BlockSpec((BN, 3 * EMBED), lambda i: (i, 0)),
        out_shape=jax.ShapeDtypeStruct((N, 3 * EMBED), jnp.float32),
    )(feats, Wqkv_t, bqkv)


def _edge_mlp_body(a_ref, w_ref, b_ref, o_ref):
    o_ref[...] = _gelu(
        jnp.dot(a_ref[...], w_ref[...], preferred_element_type=jnp.float32)
        + b_ref[...]
    )


def _edge_mlp(edge_attr, We_t, be):
    BE = 2000
    return pl.pallas_call(
        _edge_mlp_body,
        grid=(E // BE,),
        in_specs=[
            pl.BlockSpec((BE, EMBED), lambda i: (i, 0)),
            pl.BlockSpec((EMBED, EMBED), lambda i: (0, 0)),
            pl.BlockSpec((1, EMBED), lambda i: (0, 0)),
        ],
        out_specs=pl.BlockSpec((BE, EMBED), lambda i: (i, 0)),
        out_shape=jax.ShapeDtypeStruct((E, EMBED), jnp.float32),
    )(edge_attr, We_t, be)


def _final_body(f_ref, agg_ref, den_ref, eagg_ref, wo_ref, bo_ref,
                w1_ref, b1_ref, w2_ref, b2_ref, g_ref, bln_ref, o_ref):
    agg = agg_ref[...] / den_ref[...]
    att = jnp.dot(agg, wo_ref[...], preferred_element_type=jnp.float32) + bo_ref[...]
    g = g_ref[...]
    b = bln_ref[...]
    f1 = _ln(_gelu(f_ref[...] + att), g, b)
    f2 = _ln(_gelu(f1 + eagg_ref[...]), g, b)
    h = _gelu(jnp.dot(f2, w1_ref[...], preferred_element_type=jnp.float32) + b1_ref[...])
    ffn = jnp.dot(h, w2_ref[...], preferred_element_type=jnp.float32) + b2_ref[...]
    o_ref[...] = _ln(_gelu(f2 + ffn), g, b)


def _final(feats, agg_un, denom, edge_agg, Wo_t, bo, W1_t, b1, W2_t, b2, ln_g, ln_b):
    BN = 1000
    return pl.pallas_call(
        _final_body,
        grid=(N // BN,),
        in_specs=[
            pl.BlockSpec((BN, EMBED), lambda i: (i, 0)),
            pl.BlockSpec((BN, EMBED), lambda i: (i, 0)),
            pl.BlockSpec((BN, EMBED), lambda i: (i, 0)),
            pl.BlockSpec((BN, EMBED), lambda i: (i, 0)),
            pl.BlockSpec((EMBED, EMBED), lambda i: (0, 0)),
            pl.BlockSpec((1, EMBED), lambda i: (0, 0)),
            pl.BlockSpec((EMBED, DHID), lambda i: (0, 0)),
            pl.BlockSpec((1, DHID), lambda i: (0, 0)),
            pl.BlockSpec((DHID, EMBED), lambda i: (0, 0)),
            pl.BlockSpec((1, EMBED), lambda i: (0, 0)),
            pl.BlockSpec((1, EMBED), lambda i: (0, 0)),
            pl.BlockSpec((1, EMBED), lambda i: (0, 0)),
        ],
        out_specs=pl.BlockSpec((BN, EMBED), lambda i: (i, 0)),
        out_shape=jax.ShapeDtypeStruct((N, EMBED), jnp.float32),
    )(feats, agg_un, denom, edge_agg, Wo_t, bo, W1_t, b1, W2_t, b2, ln_g, ln_b)


def kernel(feats, edge_index, edge_attr, Wq, bq, Wk, bk, Wv, bv, Wo, bo,
           We, be, W1, b1, W2, b2, ln_g, ln_b):
    r = edge_index[:, 0]
    c = edge_index[:, 1]

    Wqkv_t = jnp.concatenate([Wq.T, Wk.T, Wv.T], axis=1)
    bqkv = jnp.concatenate([bq, bk, bv])[None, :]
    qkv = _qkv(feats, Wqkv_t, bqkv)
    q = qkv[:, :EMBED]
    k = qkv[:, EMBED:2 * EMBED]
    v = qkv[:, 2 * EMBED:]

    q_c = jnp.take(q, c, axis=0)
    k_r = jnp.take(k, r, axis=0)
    scores = jnp.sum((q_c * k_r).reshape(E, NHEAD, HDIM), axis=-1) / math.sqrt(HDIM)
    ex = jnp.exp(scores)
    seg_sum = jax.ops.segment_sum(ex, c, num_segments=N)
    v_r = jnp.take(v, r, axis=0).reshape(E, NHEAD, HDIM)
    weighted = (ex[:, :, None] * v_r).reshape(E, EMBED)
    agg_un = jax.ops.segment_sum(weighted, c, num_segments=N)
    denom = jnp.repeat(seg_sum + 1e-16, HDIM, axis=1)

    node_attr = _edge_mlp(edge_attr, We.T, be[None, :])
    edge_agg = jax.ops.segment_sum(node_attr, r, num_segments=N)

    out = _final(feats, agg_un, denom, edge_agg, Wo.T, bo[None, :],
                 W1.T, b1[None, :], W2.T, b2[None, :], ln_g[None, :], ln_b[None, :])
    return (out, edge_index, edge_attr)


# speedup vs baseline: 3.5133x; 1.2302x over previous
import functools
import math

import jax
import jax.numpy as jnp
from jax import lax
from jax.experimental import pallas as pl
from jax.experimental.pallas import tpu as pltpu
from jax.experimental.pallas import tpu_sc as plsc

N = 10000
E = 320000
EMBED = 128
NHEAD = 8
HDIM = EMBED // NHEAD
DHID = 4 * EMBED

NC = 2
NS = 16
NW = NC * NS
PER_W = E // NW
CB = 80
NCHUNK = PER_W // CB
NPAD = 10240
RPS = NPAD // NS
KSTAGE = 25
NSTAGE = NCHUNK // KSTAGE

_mesh = plsc.VectorSubcoreMesh(core_axis_name="c", subcore_axis_name="s")


def _ln(x, g, b):
    m = jnp.mean(x, axis=-1, keepdims=True)
    v = jnp.mean((x - m) ** 2, axis=-1, keepdims=True)
    return (x - m) * jax.lax.rsqrt(v + 1e-5) * g + b


def _gelu(x):
    return 0.5 * x * (1.0 + jax.lax.erf(x * (1.0 / math.sqrt(2.0))))


def _qkv_body(f_ref, w_ref, b_ref, q_ref, k_ref, v_ref):
    o = (
        jnp.dot(f_ref[...], w_ref[...], preferred_element_type=jnp.float32)
        + b_ref[...]
    )
    q_ref[...] = o[:, :EMBED]
    k_ref[...] = o[:, EMBED:2 * EMBED]
    v_ref[...] = o[:, 2 * EMBED:]


def _qkv(feats, Wqkv_t, bqkv):
    BN = 1000
    sds = jax.ShapeDtypeStruct((N, EMBED), jnp.float32)
    return pl.pallas_call(
        _qkv_body,
        grid=(N // BN,),
        in_specs=[
            pl.BlockSpec((BN, EMBED), lambda i: (i, 0)),
            pl.BlockSpec((EMBED, 3 * EMBED), lambda i: (0, 0)),
            pl.BlockSpec((1, 3 * EMBED), lambda i: (0, 0)),
        ],
        out_specs=[pl.BlockSpec((BN, EMBED), lambda i: (i, 0))] * 3,
        out_shape=[sds, sds, sds],
    )(feats, Wqkv_t, bqkv)


def _edge_mlp_body(a_ref, w_ref, b_ref, o_ref):
    o_ref[...] = _gelu(
        jnp.dot(a_ref[...], w_ref[...], preferred_element_type=jnp.float32)
        + b_ref[...]
    )


def _edge_mlp(edge_attr, We_t, be):
    BE = 2000
    return pl.pallas_call(
        _edge_mlp_body,
        grid=(E // BE,),
        in_specs=[
            pl.BlockSpec((BE, EMBED), lambda i: (i, 0)),
            pl.BlockSpec((EMBED, EMBED), lambda i: (0, 0)),
            pl.BlockSpec((1, EMBED), lambda i: (0, 0)),
        ],
        out_specs=pl.BlockSpec((BE, EMBED), lambda i: (i, 0)),
        out_shape=jax.ShapeDtypeStruct((E, EMBED), jnp.float32),
    )(edge_attr, We_t, be)


@functools.partial(
    pl.kernel,
    out_type=(
        jax.ShapeDtypeStruct((NC, NPAD, EMBED), jnp.float32),
        jax.ShapeDtypeStruct((NC, NPAD, NHEAD), jnp.float32),
    ),
    mesh=_mesh,
    scratch_types=[
        pltpu.VMEM((KSTAGE, CB), jnp.int32),
        pltpu.VMEM((KSTAGE, CB), jnp.int32),
        pltpu.VMEM((CB, EMBED), jnp.float32),
        pltpu.VMEM((CB, EMBED), jnp.float32),
        pltpu.VMEM((CB, EMBED), jnp.float32),
        pltpu.VMEM((CB, NHEAD), jnp.float32),
        pltpu.VMEM_SHARED((NPAD, EMBED), jnp.float32),
        pltpu.VMEM_SHARED((NPAD, NHEAD), jnp.float32),
        pltpu.SemaphoreType.DMA,
    ],
    compiler_params=pltpu.CompilerParams(use_tc_tiling_on_sc=False, needs_layout_passes=False),
)
def _sc_attention(q_hbm, k_hbm, v_hbm, r2d_hbm, c2d_hbm, z128_hbm, z8_hbm,
                  agg_out, seg_out, r_i, c_i, qb, kb, vb, exb,
                  agg_acc, seg_acc, sem):
    cid = lax.axis_index("c")
    sid = lax.axis_index("s")
    wid = cid * NS + sid
    pltpu.sync_copy(z128_hbm.at[pl.ds(sid * RPS, RPS)],
                    agg_acc.at[pl.ds(sid * RPS, RPS)])
    pltpu.sync_copy(z8_hbm.at[pl.ds(sid * RPS, RPS)],
                    seg_acc.at[pl.ds(sid * RPS, RPS)])
    plsc.subcore_barrier()

    def stage_body(s, carry0):
        pltpu.sync_copy(r2d_hbm.at[wid, pl.ds(s * KSTAGE, KSTAGE)], r_i)
        pltpu.sync_copy(c2d_hbm.at[wid, pl.ds(s * KSTAGE, KSTAGE)], c_i)

        def chunk_body(jj, carry):
            rv = r_i.at[jj]
            cv = c_i.at[jj]
            cp_q = pltpu.async_copy(q_hbm.at[cv], qb, sem)
            cp_k = pltpu.async_copy(k_hbm.at[rv], kb, sem)
            cp_v = pltpu.async_copy(v_hbm.at[rv], vb, sem)
            cp_q.wait()
            cp_k.wait()
            cp_v.wait()

            def group_body(g, carry2):
                rows = g * 16 + lax.iota(jnp.int32, 16)
                for h in range(NHEAD):
                    acc = jnp.zeros((16,), jnp.float32)
                    for d in range(HDIM):
                        col = jnp.full((16,), h * HDIM + d, jnp.int32)
                        acc = acc + (plsc.load_gather(qb, [rows, col]) *
                                     plsc.load_gather(kb, [rows, col]))
                    ex = jnp.exp(acc * (1.0 / math.sqrt(HDIM)))
                    plsc.store_scatter(exb, [rows, jnp.full((16,), h, jnp.int32)], ex)
                    for d in range(HDIM):
                        col = jnp.full((16,), h * HDIM + d, jnp.int32)
                        plsc.store_scatter(vb, [rows, col],
                                           plsc.load_gather(vb, [rows, col]) * ex)
                return carry2

            lax.fori_loop(0, CB // 16, group_body, 0)
            pltpu.sync_copy(vb, agg_acc.at[cv], add=True)
            pltpu.sync_copy(exb, seg_acc.at[cv], add=True)
            return carry

        lax.fori_loop(0, KSTAGE, chunk_body, 0)
        return carry0

    lax.fori_loop(0, NSTAGE, stage_body, 0)
    plsc.subcore_barrier()
    pltpu.sync_copy(agg_acc.at[pl.ds(sid * RPS, RPS)],
                    agg_out.at[cid, pl.ds(sid * RPS, RPS)])
    pltpu.sync_copy(seg_acc.at[pl.ds(sid * RPS, RPS)],
                    seg_out.at[cid, pl.ds(sid * RPS, RPS)])


@functools.partial(
    pl.kernel,
    out_type=jax.ShapeDtypeStruct((NC, NPAD, EMBED), jnp.float32),
    mesh=_mesh,
    scratch_types=[
        pltpu.VMEM((NCHUNK, CB), jnp.int32),
        pltpu.VMEM((CB, EMBED), jnp.float32),
        pltpu.VMEM_SHARED((NPAD, EMBED), jnp.float32),
        pltpu.SemaphoreType.DMA,
    ],
    compiler_params=pltpu.CompilerParams(use_tc_tiling_on_sc=False, needs_layout_passes=False),
)
def _sc_edge_agg(na_hbm, r2d_hbm, z128_hbm, out_hbm, r_i, nb, acc, sem):
    cid = lax.axis_index("c")
    sid = lax.axis_index("s")
    wid = cid * NS + sid
    pltpu.sync_copy(z128_hbm.at[pl.ds(sid * RPS, RPS)],
                    acc.at[pl.ds(sid * RPS, RPS)])
    pltpu.sync_copy(r2d_hbm.at[wid], r_i)
    plsc.subcore_barrier()

    def chunk_body(j, carry):
        base = wid * PER_W + j * CB
        pltpu.sync_copy(na_hbm.at[pl.ds(base, CB)], nb)
        pltpu.sync_copy(nb, acc.at[r_i.at[j]], add=True)
        return carry

    lax.fori_loop(0, NCHUNK, chunk_body, 0)
    plsc.subcore_barrier()
    pltpu.sync_copy(acc.at[pl.ds(sid * RPS, RPS)],
                    out_hbm.at[cid, pl.ds(sid * RPS, RPS)])


def _final_body(f_ref, agg_ref, seg_ref, eagg_ref, rep_ref, wo_ref, bo_ref,
                w1_ref, b1_ref, w2_ref, b2_ref, g_ref, bln_ref, o_ref):
    agg = agg_ref[0] + agg_ref[1]
    seg = seg_ref[0] + seg_ref[1]
    eagg = eagg_ref[0] + eagg_ref[1]
    den = jnp.dot(seg, rep_ref[...], preferred_element_type=jnp.float32) + 1e-16
    att = jnp.dot(agg / den, wo_ref[...], preferred_element_type=jnp.float32) \
        + bo_ref[...]
    g = g_ref[...]
    b = bln_ref[...]
    f1 = _ln(_gelu(f_ref[...] + att), g, b)
    f2 = _ln(_gelu(f1 + eagg), g, b)
    hdn = _gelu(jnp.dot(f2, w1_ref[...], preferred_element_type=jnp.float32)
                + b1_ref[...])
    ffn = jnp.dot(hdn, w2_ref[...], preferred_element_type=jnp.float32) + b2_ref[...]
    o_ref[...] = _ln(_gelu(f2 + ffn), g, b)


def _final(feats, agg2, seg2, eagg2, rep, Wo_t, bo, W1_t, b1, W2_t, b2, ln_g, ln_b):
    BN = 1000
    return pl.pallas_call(
        _final_body,
        grid=(N // BN,),
        in_specs=[
            pl.BlockSpec((BN, EMBED), lambda i: (i, 0)),
            pl.BlockSpec((NC, BN, EMBED), lambda i: (0, i, 0)),
            pl.BlockSpec((NC, BN, NHEAD), lambda i: (0, i, 0)),
            pl.BlockSpec((NC, BN, EMBED), lambda i: (0, i, 0)),
            pl.BlockSpec((NHEAD, EMBED), lambda i: (0, 0)),
            pl.BlockSpec((EMBED, EMBED), lambda i: (0, 0)),
            pl.BlockSpec((1, EMBED), lambda i: (0, 0)),
            pl.BlockSpec((EMBED, DHID), lambda i: (0, 0)),
            pl.BlockSpec((1, DHID), lambda i: (0, 0)),
            pl.BlockSpec((DHID, EMBED), lambda i: (0, 0)),
            pl.BlockSpec((1, EMBED), lambda i: (0, 0)),
            pl.BlockSpec((1, EMBED), lambda i: (0, 0)),
            pl.BlockSpec((1, EMBED), lambda i: (0, 0)),
        ],
        out_specs=pl.BlockSpec((BN, EMBED), lambda i: (i, 0)),
        out_shape=jax.ShapeDtypeStruct((N, EMBED), jnp.float32),
    )(feats, agg2, seg2, eagg2, rep, Wo_t, bo, W1_t, b1, W2_t, b2, ln_g, ln_b)


def kernel(feats, edge_index, edge_attr, Wq, bq, Wk, bk, Wv, bv, Wo, bo,
           We, be, W1, b1, W2, b2, ln_g, ln_b):
    r2d = edge_index[:, 0].reshape(NW, NCHUNK, CB)
    c2d = edge_index[:, 1].reshape(NW, NCHUNK, CB)

    Wqkv_t = jnp.concatenate([Wq.T, Wk.T, Wv.T], axis=1)
    bqkv = jnp.concatenate([bq, bk, bv])[None, :]
    q, k, v = _qkv(feats, Wqkv_t, bqkv)

    z128 = jnp.zeros((NPAD, EMBED), jnp.float32)
    z8 = jnp.zeros((NPAD, NHEAD), jnp.float32)

    agg2, seg2 = _sc_attention(q, k, v, r2d, c2d, z128, z8)

    node_attr = _edge_mlp(edge_attr, We.T, be[None, :])
    eagg2 = _sc_edge_agg(node_attr, r2d, z128)

    rep = jnp.repeat(jnp.eye(NHEAD, dtype=jnp.float32), HDIM, axis=1)

    out = _final(feats, agg2, seg2, eagg2, rep, Wo.T, bo[None, :],
                 W1.T, b1[None, :], W2.T, b2[None, :], ln_g[None, :], ln_b[None, :])
    return (out, edge_index, edge_attr)


# speedup vs baseline: 3.8892x; 1.1070x over previous
import functools
import math

import jax
import jax.numpy as jnp
from jax import lax
from jax.experimental import pallas as pl
from jax.experimental.pallas import tpu as pltpu
from jax.experimental.pallas import tpu_sc as plsc

N = 10000
E = 320000
EMBED = 128
NHEAD = 8
HDIM = EMBED // NHEAD
DHID = 4 * EMBED

NC = 2
NS = 16
NW = NC * NS
PER_W = E // NW
CB = 80
NCHUNK = PER_W // CB
NPAD = 10240
RPS = NPAD // NS
KSTAGE = 25
NSTAGE = NCHUNK // KSTAGE

_mesh = plsc.VectorSubcoreMesh(core_axis_name="c", subcore_axis_name="s")


def _ln(x, g, b):
    m = jnp.mean(x, axis=-1, keepdims=True)
    v = jnp.mean((x - m) ** 2, axis=-1, keepdims=True)
    return (x - m) * jax.lax.rsqrt(v + 1e-5) * g + b


def _gelu(x):
    return 0.5 * x * (1.0 + jax.lax.erf(x * (1.0 / math.sqrt(2.0))))


def _qkv_body(f_ref, w_ref, b_ref, q_ref, k_ref, v_ref):
    o = (
        jnp.dot(f_ref[...], w_ref[...], preferred_element_type=jnp.float32)
        + b_ref[...]
    )
    q_ref[...] = o[:, :EMBED]
    k_ref[...] = o[:, EMBED:2 * EMBED]
    v_ref[...] = o[:, 2 * EMBED:]


def _qkv(feats, Wqkv_t, bqkv):
    BN = 1000
    sds = jax.ShapeDtypeStruct((N, EMBED), jnp.float32)
    return pl.pallas_call(
        _qkv_body,
        grid=(N // BN,),
        in_specs=[
            pl.BlockSpec((BN, EMBED), lambda i: (i, 0)),
            pl.BlockSpec((EMBED, 3 * EMBED), lambda i: (0, 0)),
            pl.BlockSpec((1, 3 * EMBED), lambda i: (0, 0)),
        ],
        out_specs=[pl.BlockSpec((BN, EMBED), lambda i: (i, 0))] * 3,
        out_shape=[sds, sds, sds],
    )(feats, Wqkv_t, bqkv)


def _edge_mlp_body(a_ref, w_ref, b_ref, o_ref):
    o_ref[...] = _gelu(
        jnp.dot(a_ref[...], w_ref[...], preferred_element_type=jnp.float32)
        + b_ref[...]
    )


def _edge_mlp(edge_attr, We_t, be):
    BE = 2000
    return pl.pallas_call(
        _edge_mlp_body,
        grid=(E // BE,),
        in_specs=[
            pl.BlockSpec((BE, EMBED), lambda i: (i, 0)),
            pl.BlockSpec((EMBED, EMBED), lambda i: (0, 0)),
            pl.BlockSpec((1, EMBED), lambda i: (0, 0)),
        ],
        out_specs=pl.BlockSpec((BE, EMBED), lambda i: (i, 0)),
        out_shape=jax.ShapeDtypeStruct((E, EMBED), jnp.float32),
    )(edge_attr, We_t, be)


@functools.partial(
    pl.kernel,
    out_type=(
        jax.ShapeDtypeStruct((NC, NPAD, EMBED), jnp.float32),
        jax.ShapeDtypeStruct((NC, NPAD, 16), jnp.float32),
    ),
    mesh=_mesh,
    scratch_types=[
        pltpu.VMEM((KSTAGE, CB), jnp.int32),
        pltpu.VMEM((KSTAGE, CB), jnp.int32),
        pltpu.VMEM((CB, EMBED), jnp.float32),
        pltpu.VMEM((CB, EMBED), jnp.float32),
        pltpu.VMEM((CB, EMBED), jnp.float32),
        pltpu.VMEM((CB, 16), jnp.float32),
        pltpu.VMEM_SHARED((NPAD, EMBED), jnp.float32),
        pltpu.VMEM_SHARED((NPAD, 16), jnp.float32),
        pltpu.SemaphoreType.DMA,
    ],
    compiler_params=pltpu.CompilerParams(use_tc_tiling_on_sc=False, needs_layout_passes=False),
)
def _sc_attention(q_hbm, k_hbm, v_hbm, r2d_hbm, c2d_hbm, z128_hbm, z8_hbm,
                  agg_out, seg_out, r_i, c_i, qb, kb, vb, exb,
                  agg_acc, seg_acc, sem):
    cid = lax.axis_index("c")
    sid = lax.axis_index("s")
    wid = cid * NS + sid
    pltpu.sync_copy(z128_hbm.at[pl.ds(sid * RPS, RPS)],
                    agg_acc.at[pl.ds(sid * RPS, RPS)])
    pltpu.sync_copy(z8_hbm.at[pl.ds(sid * RPS, RPS)],
                    seg_acc.at[pl.ds(sid * RPS, RPS)])
    plsc.subcore_barrier()

    def stage_body(s, carry0):
        pltpu.sync_copy(r2d_hbm.at[wid, pl.ds(s * KSTAGE, KSTAGE)], r_i)
        pltpu.sync_copy(c2d_hbm.at[wid, pl.ds(s * KSTAGE, KSTAGE)], c_i)

        def chunk_body(jj, carry):
            rv = r_i.at[jj]
            cv = c_i.at[jj]
            cp_q = pltpu.async_copy(q_hbm.at[cv], qb, sem)
            cp_k = pltpu.async_copy(k_hbm.at[rv], kb, sem)
            cp_v = pltpu.async_copy(v_hbm.at[rv], vb, sem)
            cp_q.wait()
            cp_k.wait()
            cp_v.wait()

            iota16 = lax.iota(jnp.int32, 16)

            def edge_body(e, carry2):
                exrow = jnp.zeros((16,), jnp.float32)
                for h in range(NHEAD):
                    qv = qb[e, pl.ds(h * HDIM, HDIM)]
                    kv = kb[e, pl.ds(h * HDIM, HDIM)]
                    s = lax.reduce_sum(qv * kv, (0,)) * (1.0 / math.sqrt(HDIM))
                    exv = jnp.exp(lax.broadcast_in_dim(s, (16,), ()))
                    vb[e, pl.ds(h * HDIM, HDIM)] = (
                        vb[e, pl.ds(h * HDIM, HDIM)] * exv)
                    exrow = lax.select(iota16 == h, exv, exrow)
                plsc.store_scatter(
                    exb, [lax.broadcast_in_dim(e, (16,), ()), iota16], exrow)
                return carry2

            lax.fori_loop(0, CB, edge_body, 0)
            pltpu.sync_copy(vb, agg_acc.at[cv], add=True)
            pltpu.sync_copy(exb, seg_acc.at[cv], add=True)
            return carry

        lax.fori_loop(0, KSTAGE, chunk_body, 0)
        return carry0

    lax.fori_loop(0, NSTAGE, stage_body, 0)
    plsc.subcore_barrier()
    pltpu.sync_copy(agg_acc.at[pl.ds(sid * RPS, RPS)],
                    agg_out.at[cid, pl.ds(sid * RPS, RPS)])
    pltpu.sync_copy(seg_acc.at[pl.ds(sid * RPS, RPS)],
                    seg_out.at[cid, pl.ds(sid * RPS, RPS)])


@functools.partial(
    pl.kernel,
    out_type=jax.ShapeDtypeStruct((NC, NPAD, EMBED), jnp.float32),
    mesh=_mesh,
    scratch_types=[
        pltpu.VMEM((NCHUNK, CB), jnp.int32),
        pltpu.VMEM((CB, EMBED), jnp.float32),
        pltpu.VMEM_SHARED((NPAD, EMBED), jnp.float32),
        pltpu.SemaphoreType.DMA,
    ],
    compiler_params=pltpu.CompilerParams(use_tc_tiling_on_sc=False, needs_layout_passes=False),
)
def _sc_edge_agg(na_hbm, r2d_hbm, z128_hbm, out_hbm, r_i, nb, acc, sem):
    cid = lax.axis_index("c")
    sid = lax.axis_index("s")
    wid = cid * NS + sid
    pltpu.sync_copy(z128_hbm.at[pl.ds(sid * RPS, RPS)],
                    acc.at[pl.ds(sid * RPS, RPS)])
    pltpu.sync_copy(r2d_hbm.at[wid], r_i)
    plsc.subcore_barrier()

    def chunk_body(j, carry):
        base = wid * PER_W + j * CB
        pltpu.sync_copy(na_hbm.at[pl.ds(base, CB)], nb)
        pltpu.sync_copy(nb, acc.at[r_i.at[j]], add=True)
        return carry

    lax.fori_loop(0, NCHUNK, chunk_body, 0)
    plsc.subcore_barrier()
    pltpu.sync_copy(acc.at[pl.ds(sid * RPS, RPS)],
                    out_hbm.at[cid, pl.ds(sid * RPS, RPS)])


def _final_body(f_ref, agg_ref, seg_ref, eagg_ref, rep_ref, wo_ref, bo_ref,
                w1_ref, b1_ref, w2_ref, b2_ref, g_ref, bln_ref, o_ref):
    agg = agg_ref[0] + agg_ref[1]
    seg = seg_ref[0] + seg_ref[1]
    eagg = eagg_ref[0] + eagg_ref[1]
    den = jnp.dot(seg, rep_ref[...], preferred_element_type=jnp.float32) + 1e-16
    att = jnp.dot(agg / den, wo_ref[...], preferred_element_type=jnp.float32) \
        + bo_ref[...]
    g = g_ref[...]
    b = bln_ref[...]
    f1 = _ln(_gelu(f_ref[...] + att), g, b)
    f2 = _ln(_gelu(f1 + eagg), g, b)
    hdn = _gelu(jnp.dot(f2, w1_ref[...], preferred_element_type=jnp.float32)
                + b1_ref[...])
    ffn = jnp.dot(hdn, w2_ref[...], preferred_element_type=jnp.float32) + b2_ref[...]
    o_ref[...] = _ln(_gelu(f2 + ffn), g, b)


def _final(feats, agg2, seg2, eagg2, rep, Wo_t, bo, W1_t, b1, W2_t, b2, ln_g, ln_b):
    BN = 1000
    return pl.pallas_call(
        _final_body,
        grid=(N // BN,),
        in_specs=[
            pl.BlockSpec((BN, EMBED), lambda i: (i, 0)),
            pl.BlockSpec((NC, BN, EMBED), lambda i: (0, i, 0)),
            pl.BlockSpec((NC, BN, 16), lambda i: (0, i, 0)),
            pl.BlockSpec((NC, BN, EMBED), lambda i: (0, i, 0)),
            pl.BlockSpec((16, EMBED), lambda i: (0, 0)),
            pl.BlockSpec((EMBED, EMBED), lambda i: (0, 0)),
            pl.BlockSpec((1, EMBED), lambda i: (0, 0)),
            pl.BlockSpec((EMBED, DHID), lambda i: (0, 0)),
            pl.BlockSpec((1, DHID), lambda i: (0, 0)),
            pl.BlockSpec((DHID, EMBED), lambda i: (0, 0)),
            pl.BlockSpec((1, EMBED), lambda i: (0, 0)),
            pl.BlockSpec((1, EMBED), lambda i: (0, 0)),
            pl.BlockSpec((1, EMBED), lambda i: (0, 0)),
        ],
        out_specs=pl.BlockSpec((BN, EMBED), lambda i: (i, 0)),
        out_shape=jax.ShapeDtypeStruct((N, EMBED), jnp.float32),
    )(feats, agg2, seg2, eagg2, rep, Wo_t, bo, W1_t, b1, W2_t, b2, ln_g, ln_b)


def kernel(feats, edge_index, edge_attr, Wq, bq, Wk, bk, Wv, bv, Wo, bo,
           We, be, W1, b1, W2, b2, ln_g, ln_b):
    r2d = edge_index[:, 0].reshape(NW, NCHUNK, CB)
    c2d = edge_index[:, 1].reshape(NW, NCHUNK, CB)

    Wqkv_t = jnp.concatenate([Wq.T, Wk.T, Wv.T], axis=1)
    bqkv = jnp.concatenate([bq, bk, bv])[None, :]
    q, k, v = _qkv(feats, Wqkv_t, bqkv)

    z128 = jnp.zeros((NPAD, EMBED), jnp.float32)
    z8 = jnp.zeros((NPAD, 16), jnp.float32)

    agg2, seg2 = _sc_attention(q, k, v, r2d, c2d, z128, z8)

    node_attr = _edge_mlp(edge_attr, We.T, be[None, :])
    eagg2 = _sc_edge_agg(node_attr, r2d, z128)

    rep = jnp.concatenate(
        [jnp.repeat(jnp.eye(NHEAD, dtype=jnp.float32), HDIM, axis=1),
         jnp.zeros((16 - NHEAD, EMBED), jnp.float32)], axis=0)

    out = _final(feats, agg2, seg2, eagg2, rep, Wo.T, bo[None, :],
                 W1.T, b1[None, :], W2.T, b2[None, :], ln_g[None, :], ln_b[None, :])
    return (out, edge_index, edge_attr)


# speedup vs baseline: 11.3426x; 2.9164x over previous
import functools
import math

import jax
import jax.numpy as jnp
from jax import lax
from jax.experimental import pallas as pl
from jax.experimental.pallas import tpu as pltpu
from jax.experimental.pallas import tpu_sc as plsc

N = 10000
E = 320000
EMBED = 128
NHEAD = 8
HDIM = EMBED // NHEAD
DHID = 4 * EMBED

NC = 2
NS = 16
NW = NC * NS
PER_W = E // NW
CB = 80
NCHUNK = PER_W // CB
NPAD = 10240
RPS = NPAD // NS
KSTAGE = 25
NSTAGE = NCHUNK // KSTAGE

_mesh = plsc.VectorSubcoreMesh(core_axis_name="c", subcore_axis_name="s")


def _ln(x, g, b):
    m = jnp.mean(x, axis=-1, keepdims=True)
    v = jnp.mean((x - m) ** 2, axis=-1, keepdims=True)
    return (x - m) * jax.lax.rsqrt(v + 1e-5) * g + b


def _gelu(x):
    return 0.5 * x * (1.0 + jax.lax.erf(x * (1.0 / math.sqrt(2.0))))


def _qkv_body(f_ref, w_ref, b_ref, q_ref, k_ref, v_ref):
    o = (
        jnp.dot(f_ref[...], w_ref[...], preferred_element_type=jnp.float32)
        + b_ref[...]
    )
    q_ref[...] = o[:, :EMBED]
    k_ref[...] = o[:, EMBED:2 * EMBED]
    v_ref[...] = o[:, 2 * EMBED:]


def _qkv(feats, Wqkv_t, bqkv):
    BN = 1000
    sds = jax.ShapeDtypeStruct((N, EMBED), jnp.float32)
    return pl.pallas_call(
        _qkv_body,
        grid=(N // BN,),
        in_specs=[
            pl.BlockSpec((BN, EMBED), lambda i: (i, 0)),
            pl.BlockSpec((EMBED, 3 * EMBED), lambda i: (0, 0)),
            pl.BlockSpec((1, 3 * EMBED), lambda i: (0, 0)),
        ],
        out_specs=[pl.BlockSpec((BN, EMBED), lambda i: (i, 0))] * 3,
        out_shape=[sds, sds, sds],
    )(feats, Wqkv_t, bqkv)


def _edge_mlp_body(a_ref, w_ref, b_ref, o_ref):
    o_ref[...] = _gelu(
        jnp.dot(a_ref[...], w_ref[...], preferred_element_type=jnp.float32)
        + b_ref[...]
    )


def _edge_mlp(edge_attr, We_t, be):
    BE = 2000
    return pl.pallas_call(
        _edge_mlp_body,
        grid=(E // BE,),
        in_specs=[
            pl.BlockSpec((BE, EMBED), lambda i: (i, 0)),
            pl.BlockSpec((EMBED, EMBED), lambda i: (0, 0)),
            pl.BlockSpec((1, EMBED), lambda i: (0, 0)),
        ],
        out_specs=pl.BlockSpec((BE, EMBED), lambda i: (i, 0)),
        out_shape=jax.ShapeDtypeStruct((E, EMBED), jnp.float32),
    )(edge_attr, We_t, be)


@functools.partial(
    pl.kernel,
    out_type=(
        jax.ShapeDtypeStruct((NC, NPAD, EMBED), jnp.float32),
        jax.ShapeDtypeStruct((NC, NPAD, 16), jnp.float32),
    ),
    mesh=_mesh,
    scratch_types=[
        pltpu.VMEM((KSTAGE, CB), jnp.int32),
        pltpu.VMEM((KSTAGE, CB), jnp.int32),
        pltpu.VMEM((CB, EMBED), jnp.float32),
        pltpu.VMEM((CB, EMBED), jnp.float32),
        pltpu.VMEM((CB, EMBED), jnp.float32),
        pltpu.VMEM((CB, 16), jnp.float32),
        pltpu.VMEM_SHARED((NPAD, EMBED), jnp.float32),
        pltpu.VMEM_SHARED((NPAD, 16), jnp.float32),
        pltpu.SemaphoreType.DMA,
    ],
    compiler_params=pltpu.CompilerParams(use_tc_tiling_on_sc=False, needs_layout_passes=False),
)
def _sc_attention(q_hbm, k_hbm, v_hbm, r2d_hbm, c2d_hbm, z128_hbm, z8_hbm,
                  agg_out, seg_out, r_i, c_i, qb, kb, vb, exb,
                  agg_acc, seg_acc, sem):
    cid = lax.axis_index("c")
    sid = lax.axis_index("s")
    wid = cid * NS + sid
    pltpu.sync_copy(z128_hbm.at[pl.ds(sid * RPS, RPS)],
                    agg_acc.at[pl.ds(sid * RPS, RPS)])
    pltpu.sync_copy(z8_hbm.at[pl.ds(sid * RPS, RPS)],
                    seg_acc.at[pl.ds(sid * RPS, RPS)])
    plsc.subcore_barrier()

    def stage_body(s, carry0):
        pltpu.sync_copy(r2d_hbm.at[wid, pl.ds(s * KSTAGE, KSTAGE)], r_i)
        pltpu.sync_copy(c2d_hbm.at[wid, pl.ds(s * KSTAGE, KSTAGE)], c_i)

        def chunk_body(jj, carry):
            rv = r_i.at[jj]
            cv = c_i.at[jj]
            cp_q = pltpu.async_copy(q_hbm.at[cv], qb, sem)
            cp_k = pltpu.async_copy(k_hbm.at[rv], kb, sem)
            cp_v = pltpu.async_copy(v_hbm.at[rv], vb, sem)
            cp_q.wait()
            cp_k.wait()
            cp_v.wait()

            iota16 = lax.iota(jnp.int32, 16)

            @plsc.parallel_loop(0, CB, unroll=4)
            def edge_body(e):
                exrow = jnp.zeros((16,), jnp.float32)
                for h in range(NHEAD):
                    qv = qb[e, pl.ds(h * HDIM, HDIM)]
                    kv = kb[e, pl.ds(h * HDIM, HDIM)]
                    s = lax.reduce_sum(qv * kv, (0,)) * (1.0 / math.sqrt(HDIM))
                    exv = jnp.exp(lax.broadcast_in_dim(s, (16,), ()))
                    vb[e, pl.ds(h * HDIM, HDIM)] = (
                        vb[e, pl.ds(h * HDIM, HDIM)] * exv)
                    exrow = lax.select(iota16 == h, exv, exrow)
                plsc.store_scatter(
                    exb, [lax.broadcast_in_dim(e, (16,), ()), iota16], exrow)
            pltpu.sync_copy(vb, agg_acc.at[cv], add=True)
            pltpu.sync_copy(exb, seg_acc.at[cv], add=True)
            return carry

        lax.fori_loop(0, KSTAGE, chunk_body, 0)
        return carry0

    lax.fori_loop(0, NSTAGE, stage_body, 0)
    plsc.subcore_barrier()
    pltpu.sync_copy(agg_acc.at[pl.ds(sid * RPS, RPS)],
                    agg_out.at[cid, pl.ds(sid * RPS, RPS)])
    pltpu.sync_copy(seg_acc.at[pl.ds(sid * RPS, RPS)],
                    seg_out.at[cid, pl.ds(sid * RPS, RPS)])


@functools.partial(
    pl.kernel,
    out_type=jax.ShapeDtypeStruct((NC, NPAD, EMBED), jnp.float32),
    mesh=_mesh,
    scratch_types=[
        pltpu.VMEM((NCHUNK, CB), jnp.int32),
        pltpu.VMEM((CB, EMBED), jnp.float32),
        pltpu.VMEM_SHARED((NPAD, EMBED), jnp.float32),
        pltpu.SemaphoreType.DMA,
    ],
    compiler_params=pltpu.CompilerParams(use_tc_tiling_on_sc=False, needs_layout_passes=False),
)
def _sc_edge_agg(na_hbm, r2d_hbm, z128_hbm, out_hbm, r_i, nb, acc, sem):
    cid = lax.axis_index("c")
    sid = lax.axis_index("s")
    wid = cid * NS + sid
    pltpu.sync_copy(z128_hbm.at[pl.ds(sid * RPS, RPS)],
                    acc.at[pl.ds(sid * RPS, RPS)])
    pltpu.sync_copy(r2d_hbm.at[wid], r_i)
    plsc.subcore_barrier()

    def chunk_body(j, carry):
        base = wid * PER_W + j * CB
        pltpu.sync_copy(na_hbm.at[pl.ds(base, CB)], nb)
        pltpu.sync_copy(nb, acc.at[r_i.at[j]], add=True)
        return carry

    lax.fori_loop(0, NCHUNK, chunk_body, 0)
    plsc.subcore_barrier()
    pltpu.sync_copy(acc.at[pl.ds(sid * RPS, RPS)],
                    out_hbm.at[cid, pl.ds(sid * RPS, RPS)])


def _final_body(f_ref, agg_ref, seg_ref, eagg_ref, rep_ref, wo_ref, bo_ref,
                w1_ref, b1_ref, w2_ref, b2_ref, g_ref, bln_ref, o_ref):
    agg = agg_ref[0] + agg_ref[1]
    seg = seg_ref[0] + seg_ref[1]
    eagg = eagg_ref[0] + eagg_ref[1]
    den = jnp.dot(seg, rep_ref[...], preferred_element_type=jnp.float32) + 1e-16
    att = jnp.dot(agg / den, wo_ref[...], preferred_element_type=jnp.float32) \
        + bo_ref[...]
    g = g_ref[...]
    b = bln_ref[...]
    f1 = _ln(_gelu(f_ref[...] + att), g, b)
    f2 = _ln(_gelu(f1 + eagg), g, b)
    hdn = _gelu(jnp.dot(f2, w1_ref[...], preferred_element_type=jnp.float32)
                + b1_ref[...])
    ffn = jnp.dot(hdn, w2_ref[...], preferred_element_type=jnp.float32) + b2_ref[...]
    o_ref[...] = _ln(_gelu(f2 + ffn), g, b)


def _final(feats, agg2, seg2, eagg2, rep, Wo_t, bo, W1_t, b1, W2_t, b2, ln_g, ln_b):
    BN = 1000
    return pl.pallas_call(
        _final_body,
        grid=(N // BN,),
        in_specs=[
            pl.BlockSpec((BN, EMBED), lambda i: (i, 0)),
            pl.BlockSpec((NC, BN, EMBED), lambda i: (0, i, 0)),
            pl.BlockSpec((NC, BN, 16), lambda i: (0, i, 0)),
            pl.BlockSpec((NC, BN, EMBED), lambda i: (0, i, 0)),
            pl.BlockSpec((16, EMBED), lambda i: (0, 0)),
            pl.BlockSpec((EMBED, EMBED), lambda i: (0, 0)),
            pl.BlockSpec((1, EMBED), lambda i: (0, 0)),
            pl.BlockSpec((EMBED, DHID), lambda i: (0, 0)),
            pl.BlockSpec((1, DHID), lambda i: (0, 0)),
            pl.BlockSpec((DHID, EMBED), lambda i: (0, 0)),
            pl.BlockSpec((1, EMBED), lambda i: (0, 0)),
            pl.BlockSpec((1, EMBED), lambda i: (0, 0)),
            pl.BlockSpec((1, EMBED), lambda i: (0, 0)),
        ],
        out_specs=pl.BlockSpec((BN, EMBED), lambda i: (i, 0)),
        out_shape=jax.ShapeDtypeStruct((N, EMBED), jnp.float32),
    )(feats, agg2, seg2, eagg2, rep, Wo_t, bo, W1_t, b1, W2_t, b2, ln_g, ln_b)


def kernel(feats, edge_index, edge_attr, Wq, bq, Wk, bk, Wv, bv, Wo, bo,
           We, be, W1, b1, W2, b2, ln_g, ln_b):
    r2d = edge_index[:, 0].reshape(NW, NCHUNK, CB)
    c2d = edge_index[:, 1].reshape(NW, NCHUNK, CB)

    Wqkv_t = jnp.concatenate([Wq.T, Wk.T, Wv.T], axis=1)
    bqkv = jnp.concatenate([bq, bk, bv])[None, :]
    q, k, v = _qkv(feats, Wqkv_t, bqkv)

    z128 = jnp.zeros((NPAD, EMBED), jnp.float32)
    z8 = jnp.zeros((NPAD, 16), jnp.float32)

    agg2, seg2 = _sc_attention(q, k, v, r2d, c2d, z128, z8)

    node_attr = _edge_mlp(edge_attr, We.T, be[None, :])
    eagg2 = _sc_edge_agg(node_attr, r2d, z128)

    rep = jnp.concatenate(
        [jnp.repeat(jnp.eye(NHEAD, dtype=jnp.float32), HDIM, axis=1),
         jnp.zeros((16 - NHEAD, EMBED), jnp.float32)], axis=0)

    out = _final(feats, agg2, seg2, eagg2, rep, Wo.T, bo[None, :],
                 W1.T, b1[None, :], W2.T, b2[None, :], ln_g[None, :], ln_b[None, :])
    return (out, edge_index, edge_attr)


# speedup vs baseline: 13.4796x; 1.1884x over previous
import functools
import math

import jax
import jax.numpy as jnp
from jax import lax
from jax.experimental import pallas as pl
from jax.experimental.pallas import tpu as pltpu
from jax.experimental.pallas import tpu_sc as plsc

N = 10000
E = 320000
EMBED = 128
NHEAD = 8
HDIM = EMBED // NHEAD
DHID = 4 * EMBED

NC = 2
NS = 16
NW = NC * NS
PER_W = E // NW
CB = 80
NCHUNK = PER_W // CB
NPAD = 10240
RPS = NPAD // NS
KSTAGE = 25
NSTAGE = NCHUNK // KSTAGE

_mesh = plsc.VectorSubcoreMesh(core_axis_name="c", subcore_axis_name="s")


def _ln(x, g, b):
    m = jnp.mean(x, axis=-1, keepdims=True)
    v = jnp.mean((x - m) ** 2, axis=-1, keepdims=True)
    return (x - m) * jax.lax.rsqrt(v + 1e-5) * g + b


def _gelu(x):
    return 0.5 * x * (1.0 + jax.lax.erf(x * (1.0 / math.sqrt(2.0))))


def _qkv_body(f_ref, w_ref, b_ref, q_ref, k_ref, v_ref):
    o = (
        jnp.dot(f_ref[...], w_ref[...], preferred_element_type=jnp.float32)
        + b_ref[...]
    )
    q_ref[...] = o[:, :EMBED]
    k_ref[...] = o[:, EMBED:2 * EMBED]
    v_ref[...] = o[:, 2 * EMBED:]


def _qkv(feats, Wqkv_t, bqkv):
    BN = 1000
    sds = jax.ShapeDtypeStruct((N, EMBED), jnp.float32)
    return pl.pallas_call(
        _qkv_body,
        grid=(N // BN,),
        in_specs=[
            pl.BlockSpec((BN, EMBED), lambda i: (i, 0)),
            pl.BlockSpec((EMBED, 3 * EMBED), lambda i: (0, 0)),
            pl.BlockSpec((1, 3 * EMBED), lambda i: (0, 0)),
        ],
        out_specs=[pl.BlockSpec((BN, EMBED), lambda i: (i, 0))] * 3,
        out_shape=[sds, sds, sds],
    )(feats, Wqkv_t, bqkv)


def _edge_mlp_body(a_ref, w_ref, b_ref, o_ref):
    o_ref[...] = _gelu(
        jnp.dot(a_ref[...], w_ref[...], preferred_element_type=jnp.float32)
        + b_ref[...]
    )


def _edge_mlp(edge_attr, We_t, be):
    BE = 2000
    return pl.pallas_call(
        _edge_mlp_body,
        grid=(E // BE,),
        in_specs=[
            pl.BlockSpec((BE, EMBED), lambda i: (i, 0)),
            pl.BlockSpec((EMBED, EMBED), lambda i: (0, 0)),
            pl.BlockSpec((1, EMBED), lambda i: (0, 0)),
        ],
        out_specs=pl.BlockSpec((BE, EMBED), lambda i: (i, 0)),
        out_shape=jax.ShapeDtypeStruct((E, EMBED), jnp.float32),
    )(edge_attr, We_t, be)


@functools.partial(
    pl.kernel,
    out_type=(
        jax.ShapeDtypeStruct((NC, NPAD, EMBED), jnp.float32),
        jax.ShapeDtypeStruct((NC, NPAD, 16), jnp.float32),
    ),
    mesh=_mesh,
    scratch_types=[
        pltpu.VMEM((KSTAGE, CB), jnp.int32),
        pltpu.VMEM((KSTAGE, CB), jnp.int32),
        pltpu.VMEM((CB, EMBED), jnp.float32),
        pltpu.VMEM((CB, EMBED), jnp.float32),
        pltpu.VMEM((CB, EMBED), jnp.float32),
        pltpu.VMEM((CB, 16), jnp.float32),
        pltpu.VMEM_SHARED((NPAD, EMBED), jnp.float32),
        pltpu.VMEM_SHARED((NPAD, 16), jnp.float32),
        pltpu.SemaphoreType.DMA,
    ],
    compiler_params=pltpu.CompilerParams(use_tc_tiling_on_sc=False, needs_layout_passes=False),
)
def _sc_attention(q_hbm, k_hbm, v_hbm, r2d_hbm, c2d_hbm, z128_hbm, z8_hbm,
                  agg_out, seg_out, r_i, c_i, qb, kb, vb, exb,
                  agg_acc, seg_acc, sem):
    cid = lax.axis_index("c")
    sid = lax.axis_index("s")
    wid = cid * NS + sid
    pltpu.sync_copy(z128_hbm.at[pl.ds(sid * RPS, RPS)],
                    agg_acc.at[pl.ds(sid * RPS, RPS)])
    pltpu.sync_copy(z8_hbm.at[pl.ds(sid * RPS, RPS)],
                    seg_acc.at[pl.ds(sid * RPS, RPS)])
    plsc.subcore_barrier()

    def stage_body(s, carry0):
        pltpu.sync_copy(r2d_hbm.at[wid, pl.ds(s * KSTAGE, KSTAGE)], r_i)
        pltpu.sync_copy(c2d_hbm.at[wid, pl.ds(s * KSTAGE, KSTAGE)], c_i)

        def chunk_body(jj, carry):
            rv = r_i.at[jj]
            cv = c_i.at[jj]
            cp_q = pltpu.async_copy(q_hbm.at[cv], qb, sem)
            cp_k = pltpu.async_copy(k_hbm.at[rv], kb, sem)
            cp_v = pltpu.async_copy(v_hbm.at[rv], vb, sem)
            cp_q.wait()
            cp_k.wait()
            cp_v.wait()

            iota16 = lax.iota(jnp.int32, 16)

            @plsc.parallel_loop(0, CB, unroll=8)
            def edge_body(e):
                exrow = jnp.zeros((16,), jnp.float32)
                for h in range(NHEAD):
                    qv = qb[e, pl.ds(h * HDIM, HDIM)]
                    kv = kb[e, pl.ds(h * HDIM, HDIM)]
                    s = lax.reduce_sum(qv * kv, (0,)) * (1.0 / math.sqrt(HDIM))
                    exv = jnp.exp(lax.broadcast_in_dim(s, (16,), ()))
                    vb[e, pl.ds(h * HDIM, HDIM)] = (
                        vb[e, pl.ds(h * HDIM, HDIM)] * exv)
                    exrow = lax.select(iota16 == h, exv, exrow)
                plsc.store_scatter(
                    exb, [lax.broadcast_in_dim(e, (16,), ()), iota16], exrow)
            pltpu.sync_copy(vb, agg_acc.at[cv], add=True)
            pltpu.sync_copy(exb, seg_acc.at[cv], add=True)
            return carry

        lax.fori_loop(0, KSTAGE, chunk_body, 0)
        return carry0

    lax.fori_loop(0, NSTAGE, stage_body, 0)
    plsc.subcore_barrier()
    pltpu.sync_copy(agg_acc.at[pl.ds(sid * RPS, RPS)],
                    agg_out.at[cid, pl.ds(sid * RPS, RPS)])
    pltpu.sync_copy(seg_acc.at[pl.ds(sid * RPS, RPS)],
                    seg_out.at[cid, pl.ds(sid * RPS, RPS)])


@functools.partial(
    pl.kernel,
    out_type=jax.ShapeDtypeStruct((NC, NPAD, EMBED), jnp.float32),
    mesh=_mesh,
    scratch_types=[
        pltpu.VMEM((NCHUNK, CB), jnp.int32),
        pltpu.VMEM((CB, EMBED), jnp.float32),
        pltpu.VMEM_SHARED((NPAD, EMBED), jnp.float32),
        pltpu.SemaphoreType.DMA,
    ],
    compiler_params=pltpu.CompilerParams(use_tc_tiling_on_sc=False, needs_layout_passes=False),
)
def _sc_edge_agg(na_hbm, r2d_hbm, z128_hbm, out_hbm, r_i, nb, acc, sem):
    cid = lax.axis_index("c")
    sid = lax.axis_index("s")
    wid = cid * NS + sid
    pltpu.sync_copy(z128_hbm.at[pl.ds(sid * RPS, RPS)],
                    acc.at[pl.ds(sid * RPS, RPS)])
    pltpu.sync_copy(r2d_hbm.at[wid], r_i)
    plsc.subcore_barrier()

    def chunk_body(j, carry):
        base = wid * PER_W + j * CB
        pltpu.sync_copy(na_hbm.at[pl.ds(base, CB)], nb)
        pltpu.sync_copy(nb, acc.at[r_i.at[j]], add=True)
        return carry

    lax.fori_loop(0, NCHUNK, chunk_body, 0)
    plsc.subcore_barrier()
    pltpu.sync_copy(acc.at[pl.ds(sid * RPS, RPS)],
                    out_hbm.at[cid, pl.ds(sid * RPS, RPS)])


def _final_body(f_ref, agg_ref, seg_ref, eagg_ref, rep_ref, wo_ref, bo_ref,
                w1_ref, b1_ref, w2_ref, b2_ref, g_ref, bln_ref, o_ref):
    agg = agg_ref[0] + agg_ref[1]
    seg = seg_ref[0] + seg_ref[1]
    eagg = eagg_ref[0] + eagg_ref[1]
    den = jnp.dot(seg, rep_ref[...], preferred_element_type=jnp.float32) + 1e-16
    att = jnp.dot(agg / den, wo_ref[...], preferred_element_type=jnp.float32) \
        + bo_ref[...]
    g = g_ref[...]
    b = bln_ref[...]
    f1 = _ln(_gelu(f_ref[...] + att), g, b)
    f2 = _ln(_gelu(f1 + eagg), g, b)
    hdn = _gelu(jnp.dot(f2, w1_ref[...], preferred_element_type=jnp.float32)
                + b1_ref[...])
    ffn = jnp.dot(hdn, w2_ref[...], preferred_element_type=jnp.float32) + b2_ref[...]
    o_ref[...] = _ln(_gelu(f2 + ffn), g, b)


def _final(feats, agg2, seg2, eagg2, rep, Wo_t, bo, W1_t, b1, W2_t, b2, ln_g, ln_b):
    BN = 1000
    return pl.pallas_call(
        _final_body,
        grid=(N // BN,),
        in_specs=[
            pl.BlockSpec((BN, EMBED), lambda i: (i, 0)),
            pl.BlockSpec((NC, BN, EMBED), lambda i: (0, i, 0)),
            pl.BlockSpec((NC, BN, 16), lambda i: (0, i, 0)),
            pl.BlockSpec((NC, BN, EMBED), lambda i: (0, i, 0)),
            pl.BlockSpec((16, EMBED), lambda i: (0, 0)),
            pl.BlockSpec((EMBED, EMBED), lambda i: (0, 0)),
            pl.BlockSpec((1, EMBED), lambda i: (0, 0)),
            pl.BlockSpec((EMBED, DHID), lambda i: (0, 0)),
            pl.BlockSpec((1, DHID), lambda i: (0, 0)),
            pl.BlockSpec((DHID, EMBED), lambda i: (0, 0)),
            pl.BlockSpec((1, EMBED), lambda i: (0, 0)),
            pl.BlockSpec((1, EMBED), lambda i: (0, 0)),
            pl.BlockSpec((1, EMBED), lambda i: (0, 0)),
        ],
        out_specs=pl.BlockSpec((BN, EMBED), lambda i: (i, 0)),
        out_shape=jax.ShapeDtypeStruct((N, EMBED), jnp.float32),
    )(feats, agg2, seg2, eagg2, rep, Wo_t, bo, W1_t, b1, W2_t, b2, ln_g, ln_b)


def kernel(feats, edge_index, edge_attr, Wq, bq, Wk, bk, Wv, bv, Wo, bo,
           We, be, W1, b1, W2, b2, ln_g, ln_b):
    r2d = edge_index[:, 0].reshape(NW, NCHUNK, CB)
    c2d = edge_index[:, 1].reshape(NW, NCHUNK, CB)

    Wqkv_t = jnp.concatenate([Wq.T, Wk.T, Wv.T], axis=1)
    bqkv = jnp.concatenate([bq, bk, bv])[None, :]
    q, k, v = _qkv(feats, Wqkv_t, bqkv)

    z128 = jnp.zeros((NPAD, EMBED), jnp.float32)
    z8 = jnp.zeros((NPAD, 16), jnp.float32)

    agg2, seg2 = _sc_attention(q, k, v, r2d, c2d, z128, z8)

    node_attr = _edge_mlp(edge_attr, We.T, be[None, :])
    eagg2 = _sc_edge_agg(node_attr, r2d, z128)

    rep = jnp.concatenate(
        [jnp.repeat(jnp.eye(NHEAD, dtype=jnp.float32), HDIM, axis=1),
         jnp.zeros((16 - NHEAD, EMBED), jnp.float32)], axis=0)

    out = _final(feats, agg2, seg2, eagg2, rep, Wo.T, bo[None, :],
                 W1.T, b1[None, :], W2.T, b2[None, :], ln_g[None, :], ln_b[None, :])
    return (out, edge_index, edge_attr)


# speedup vs baseline: 14.8648x; 1.1028x over previous
import functools
import math

import jax
import jax.numpy as jnp
from jax import lax
from jax.experimental import pallas as pl
from jax.experimental.pallas import tpu as pltpu
from jax.experimental.pallas import tpu_sc as plsc

N = 10000
E = 320000
EMBED = 128
NHEAD = 8
HDIM = EMBED // NHEAD
DHID = 4 * EMBED

NC = 2
NS = 16
NW = NC * NS
PER_W = E // NW
CB = 80
NCHUNK = PER_W // CB
NPAD = 10240
RPS = NPAD // NS
KSTAGE = 25
NSTAGE = NCHUNK // KSTAGE

_mesh = plsc.VectorSubcoreMesh(core_axis_name="c", subcore_axis_name="s")


def _ln(x, g, b):
    m = jnp.mean(x, axis=-1, keepdims=True)
    v = jnp.mean((x - m) ** 2, axis=-1, keepdims=True)
    return (x - m) * jax.lax.rsqrt(v + 1e-5) * g + b


def _gelu(x):
    return 0.5 * x * (1.0 + jax.lax.erf(x * (1.0 / math.sqrt(2.0))))


def _qkv_body(f_ref, w_ref, b_ref, q_ref, k_ref, v_ref):
    o = (
        jnp.dot(f_ref[...], w_ref[...], preferred_element_type=jnp.float32)
        + b_ref[...]
    )
    q_ref[...] = o[:, :EMBED]
    k_ref[...] = o[:, EMBED:2 * EMBED]
    v_ref[...] = o[:, 2 * EMBED:]


def _qkv(feats, Wqkv_t, bqkv):
    BN = 1000
    sds = jax.ShapeDtypeStruct((N, EMBED), jnp.float32)
    return pl.pallas_call(
        _qkv_body,
        grid=(N // BN,),
        in_specs=[
            pl.BlockSpec((BN, EMBED), lambda i: (i, 0)),
            pl.BlockSpec((EMBED, 3 * EMBED), lambda i: (0, 0)),
            pl.BlockSpec((1, 3 * EMBED), lambda i: (0, 0)),
        ],
        out_specs=[pl.BlockSpec((BN, EMBED), lambda i: (i, 0))] * 3,
        out_shape=[sds, sds, sds],
    )(feats, Wqkv_t, bqkv)


def _edge_mlp_body(a_ref, w_ref, b_ref, o_ref):
    o_ref[...] = _gelu(
        jnp.dot(a_ref[...], w_ref[...], preferred_element_type=jnp.float32)
        + b_ref[...]
    )


def _edge_mlp(edge_attr, We_t, be):
    BE = 2000
    return pl.pallas_call(
        _edge_mlp_body,
        grid=(E // BE,),
        in_specs=[
            pl.BlockSpec((BE, EMBED), lambda i: (i, 0)),
            pl.BlockSpec((EMBED, EMBED), lambda i: (0, 0)),
            pl.BlockSpec((1, EMBED), lambda i: (0, 0)),
        ],
        out_specs=pl.BlockSpec((BE, EMBED), lambda i: (i, 0)),
        out_shape=jax.ShapeDtypeStruct((E, EMBED), jnp.float32),
    )(edge_attr, We_t, be)


@functools.partial(
    pl.kernel,
    out_type=(
        jax.ShapeDtypeStruct((NC, NPAD, EMBED), jnp.float32),
        jax.ShapeDtypeStruct((NC, NPAD, 16), jnp.float32),
    ),
    mesh=_mesh,
    scratch_types=[
        pltpu.VMEM((KSTAGE, CB), jnp.int32),
        pltpu.VMEM((KSTAGE, CB), jnp.int32),
        pltpu.VMEM((CB, EMBED), jnp.float32),
        pltpu.VMEM((CB, EMBED), jnp.float32),
        pltpu.VMEM((CB, EMBED), jnp.float32),
        pltpu.VMEM((CB, 16), jnp.float32),
        pltpu.VMEM_SHARED((NPAD, EMBED), jnp.float32),
        pltpu.VMEM_SHARED((NPAD, 16), jnp.float32),
        pltpu.SemaphoreType.DMA,
    ],
    compiler_params=pltpu.CompilerParams(use_tc_tiling_on_sc=False, needs_layout_passes=False),
)
def _sc_attention(q_hbm, k_hbm, v_hbm, r2d_hbm, c2d_hbm, z128_hbm, z8_hbm,
                  agg_out, seg_out, r_i, c_i, qb, kb, vb, exb,
                  agg_acc, seg_acc, sem):
    cid = lax.axis_index("c")
    sid = lax.axis_index("s")
    wid = cid * NS + sid
    pltpu.sync_copy(z128_hbm.at[pl.ds(sid * RPS, RPS)],
                    agg_acc.at[pl.ds(sid * RPS, RPS)])
    pltpu.sync_copy(z8_hbm.at[pl.ds(sid * RPS, RPS)],
                    seg_acc.at[pl.ds(sid * RPS, RPS)])
    plsc.subcore_barrier()

    def stage_body(s, carry0):
        pltpu.sync_copy(r2d_hbm.at[wid, pl.ds(s * KSTAGE, KSTAGE)], r_i)
        pltpu.sync_copy(c2d_hbm.at[wid, pl.ds(s * KSTAGE, KSTAGE)], c_i)

        def chunk_body(jj, carry):
            rv = r_i.at[jj]
            cv = c_i.at[jj]
            cp_q = pltpu.async_copy(q_hbm.at[cv], qb, sem)
            cp_k = pltpu.async_copy(k_hbm.at[rv], kb, sem)
            cp_v = pltpu.async_copy(v_hbm.at[rv], vb, sem)
            cp_q.wait()
            cp_k.wait()
            cp_v.wait()

            iota16 = lax.iota(jnp.int32, 16)

            @plsc.parallel_loop(0, CB, unroll=8)
            def edge_body(e):
                exrow = jnp.zeros((16,), jnp.float32)
                for h in range(NHEAD):
                    qv = qb[e, pl.ds(h * HDIM, HDIM)]
                    kv = kb[e, pl.ds(h * HDIM, HDIM)]
                    s = lax.reduce_sum(qv * kv, (0,)) * (1.0 / math.sqrt(HDIM))
                    exv = jnp.exp(lax.broadcast_in_dim(s, (16,), ()))
                    vb[e, pl.ds(h * HDIM, HDIM)] = (
                        vb[e, pl.ds(h * HDIM, HDIM)] * exv)
                    exrow = lax.select(iota16 == h, exv, exrow)
                plsc.store_scatter(
                    exb, [lax.broadcast_in_dim(e, (16,), ()), iota16], exrow)
            pltpu.sync_copy(vb, agg_acc.at[cv], add=True)
            pltpu.sync_copy(exb, seg_acc.at[cv], add=True)
            return carry

        lax.fori_loop(0, KSTAGE, chunk_body, 0)
        return carry0

    lax.fori_loop(0, NSTAGE, stage_body, 0)
    plsc.subcore_barrier()
    pltpu.sync_copy(agg_acc.at[pl.ds(sid * RPS, RPS)],
                    agg_out.at[cid, pl.ds(sid * RPS, RPS)])
    pltpu.sync_copy(seg_acc.at[pl.ds(sid * RPS, RPS)],
                    seg_out.at[cid, pl.ds(sid * RPS, RPS)])


@functools.partial(
    pl.kernel,
    out_type=jax.ShapeDtypeStruct((NC, NPAD, EMBED), jnp.float32),
    mesh=_mesh,
    scratch_types=[
        pltpu.VMEM((NCHUNK, CB), jnp.int32),
        pltpu.VMEM((CB, EMBED), jnp.float32),
        pltpu.VMEM((CB, EMBED), jnp.float32),
        pltpu.VMEM_SHARED((NPAD, EMBED), jnp.float32),
        pltpu.SemaphoreType.DMA,
        pltpu.SemaphoreType.DMA,
    ],
    compiler_params=pltpu.CompilerParams(use_tc_tiling_on_sc=False, needs_layout_passes=False),
)
def _sc_edge_agg(na_hbm, r2d_hbm, z128_hbm, out_hbm, r_i, nb0, nb1, acc,
                 sem0, sem1):
    cid = lax.axis_index("c")
    sid = lax.axis_index("s")
    wid = cid * NS + sid
    pltpu.sync_copy(z128_hbm.at[pl.ds(sid * RPS, RPS)],
                    acc.at[pl.ds(sid * RPS, RPS)])
    pltpu.sync_copy(r2d_hbm.at[wid], r_i)
    plsc.subcore_barrier()

    nbufs = (nb0, nb1)
    sems = (sem0, sem1)
    base0 = wid * PER_W

    pltpu.async_copy(na_hbm.at[pl.ds(base0, CB)], nb0, sem0)
    pltpu.async_copy(na_hbm.at[pl.ds(base0 + CB, CB)], nb1, sem1)

    def pair_body(j2, carry):
        for b in range(2):
            t = j2 * 2 + b
            pltpu.make_async_copy(na_hbm.at[pl.ds(0, CB)], nbufs[b],
                                  sems[b]).wait()
            pltpu.sync_copy(nbufs[b], acc.at[r_i.at[t]], add=True)

            @pl.when(t + 2 <= NCHUNK - 1)
            def _():
                pltpu.async_copy(na_hbm.at[pl.ds(base0 + (t + 2) * CB, CB)],
                                 nbufs[b], sems[b])
        return carry

    lax.fori_loop(0, (NCHUNK - 1) // 2, pair_body, 0)
    pltpu.make_async_copy(na_hbm.at[pl.ds(0, CB)], nb0, sem0).wait()
    pltpu.sync_copy(nb0, acc.at[r_i.at[NCHUNK - 1]], add=True)
    plsc.subcore_barrier()
    pltpu.sync_copy(acc.at[pl.ds(sid * RPS, RPS)],
                    out_hbm.at[cid, pl.ds(sid * RPS, RPS)])


def _final_body(f_ref, agg_ref, seg_ref, eagg_ref, rep_ref, wo_ref, bo_ref,
                w1_ref, b1_ref, w2_ref, b2_ref, g_ref, bln_ref, o_ref):
    agg = agg_ref[0] + agg_ref[1]
    seg = seg_ref[0] + seg_ref[1]
    eagg = eagg_ref[0] + eagg_ref[1]
    den = jnp.dot(seg, rep_ref[...], preferred_element_type=jnp.float32) + 1e-16
    att = jnp.dot(agg / den, wo_ref[...], preferred_element_type=jnp.float32) \
        + bo_ref[...]
    g = g_ref[...]
    b = bln_ref[...]
    f1 = _ln(_gelu(f_ref[...] + att), g, b)
    f2 = _ln(_gelu(f1 + eagg), g, b)
    hdn = _gelu(jnp.dot(f2, w1_ref[...], preferred_element_type=jnp.float32)
                + b1_ref[...])
    ffn = jnp.dot(hdn, w2_ref[...], preferred_element_type=jnp.float32) + b2_ref[...]
    o_ref[...] = _ln(_gelu(f2 + ffn), g, b)


def _final(feats, agg2, seg2, eagg2, rep, Wo_t, bo, W1_t, b1, W2_t, b2, ln_g, ln_b):
    BN = 1000
    return pl.pallas_call(
        _final_body,
        grid=(N // BN,),
        in_specs=[
            pl.BlockSpec((BN, EMBED), lambda i: (i, 0)),
            pl.BlockSpec((NC, BN, EMBED), lambda i: (0, i, 0)),
            pl.BlockSpec((NC, BN, 16), lambda i: (0, i, 0)),
            pl.BlockSpec((NC, BN, EMBED), lambda i: (0, i, 0)),
            pl.BlockSpec((16, EMBED), lambda i: (0, 0)),
            pl.BlockSpec((EMBED, EMBED), lambda i: (0, 0)),
            pl.BlockSpec((1, EMBED), lambda i: (0, 0)),
            pl.BlockSpec((EMBED, DHID), lambda i: (0, 0)),
            pl.BlockSpec((1, DHID), lambda i: (0, 0)),
            pl.BlockSpec((DHID, EMBED), lambda i: (0, 0)),
            pl.BlockSpec((1, EMBED), lambda i: (0, 0)),
            pl.BlockSpec((1, EMBED), lambda i: (0, 0)),
            pl.BlockSpec((1, EMBED), lambda i: (0, 0)),
        ],
        out_specs=pl.BlockSpec((BN, EMBED), lambda i: (i, 0)),
        out_shape=jax.ShapeDtypeStruct((N, EMBED), jnp.float32),
    )(feats, agg2, seg2, eagg2, rep, Wo_t, bo, W1_t, b1, W2_t, b2, ln_g, ln_b)


def kernel(feats, edge_index, edge_attr, Wq, bq, Wk, bk, Wv, bv, Wo, bo,
           We, be, W1, b1, W2, b2, ln_g, ln_b):
    r2d = edge_index[:, 0].reshape(NW, NCHUNK, CB)
    c2d = edge_index[:, 1].reshape(NW, NCHUNK, CB)

    Wqkv_t = jnp.concatenate([Wq.T, Wk.T, Wv.T], axis=1)
    bqkv = jnp.concatenate([bq, bk, bv])[None, :]
    q, k, v = _qkv(feats, Wqkv_t, bqkv)

    z128 = jnp.zeros((NPAD, EMBED), jnp.float32)
    z8 = jnp.zeros((NPAD, 16), jnp.float32)

    agg2, seg2 = _sc_attention(q, k, v, r2d, c2d, z128, z8)

    node_attr = _edge_mlp(edge_attr, We.T, be[None, :])
    eagg2 = _sc_edge_agg(node_attr, r2d, z128)

    rep = jnp.concatenate(
        [jnp.repeat(jnp.eye(NHEAD, dtype=jnp.float32), HDIM, axis=1),
         jnp.zeros((16 - NHEAD, EMBED), jnp.float32)], axis=0)

    out = _final(feats, agg2, seg2, eagg2, rep, Wo.T, bo[None, :],
                 W1.T, b1[None, :], W2.T, b2[None, :], ln_g[None, :], ln_b[None, :])
    return (out, edge_index, edge_attr)


# speedup vs baseline: 16.0407x; 1.0791x over previous
import functools
import math

import jax
import jax.numpy as jnp
from jax import lax
from jax.experimental import pallas as pl
from jax.experimental.pallas import tpu as pltpu
from jax.experimental.pallas import tpu_sc as plsc

N = 10000
E = 320000
EMBED = 128
NHEAD = 8
HDIM = EMBED // NHEAD
DHID = 4 * EMBED

NC = 2
NS = 16
NW = NC * NS
PER_W = E // NW
CB = 80
NCHUNK = PER_W // CB
NPAD = 10240
RPS = NPAD // NS
KSTAGE = 25
NSTAGE = NCHUNK // KSTAGE

_mesh = plsc.VectorSubcoreMesh(core_axis_name="c", subcore_axis_name="s")


def _ln(x, g, b):
    m = jnp.mean(x, axis=-1, keepdims=True)
    v = jnp.mean((x - m) ** 2, axis=-1, keepdims=True)
    return (x - m) * jax.lax.rsqrt(v + 1e-5) * g + b


def _gelu(x):
    return 0.5 * x * (1.0 + jax.lax.erf(x * (1.0 / math.sqrt(2.0))))


def _qkv_body(f_ref, w_ref, b_ref, q_ref, k_ref, v_ref):
    o = (
        jnp.dot(f_ref[...], w_ref[...], preferred_element_type=jnp.float32)
        + b_ref[...]
    )
    q_ref[...] = o[:, :EMBED]
    k_ref[...] = o[:, EMBED:2 * EMBED]
    v_ref[...] = o[:, 2 * EMBED:]


def _qkv(feats, Wqkv_t, bqkv):
    BN = 1000
    sds = jax.ShapeDtypeStruct((N, EMBED), jnp.float32)
    return pl.pallas_call(
        _qkv_body,
        grid=(N // BN,),
        in_specs=[
            pl.BlockSpec((BN, EMBED), lambda i: (i, 0)),
            pl.BlockSpec((EMBED, 3 * EMBED), lambda i: (0, 0)),
            pl.BlockSpec((1, 3 * EMBED), lambda i: (0, 0)),
        ],
        out_specs=[pl.BlockSpec((BN, EMBED), lambda i: (i, 0))] * 3,
        out_shape=[sds, sds, sds],
    )(feats, Wqkv_t, bqkv)


def _edge_mlp_body(a_ref, w_ref, b_ref, o_ref):
    o_ref[...] = _gelu(
        jnp.dot(a_ref[...], w_ref[...], preferred_element_type=jnp.float32)
        + b_ref[...]
    )


def _edge_mlp(edge_attr, We_t, be):
    BE = 2000
    return pl.pallas_call(
        _edge_mlp_body,
        grid=(E // BE,),
        in_specs=[
            pl.BlockSpec((BE, EMBED), lambda i: (i, 0)),
            pl.BlockSpec((EMBED, EMBED), lambda i: (0, 0)),
            pl.BlockSpec((1, EMBED), lambda i: (0, 0)),
        ],
        out_specs=pl.BlockSpec((BE, EMBED), lambda i: (i, 0)),
        out_shape=jax.ShapeDtypeStruct((E, EMBED), jnp.float32),
    )(edge_attr, We_t, be)


@functools.partial(
    pl.kernel,
    out_type=(
        jax.ShapeDtypeStruct((NC, NPAD, EMBED), jnp.float32),
        jax.ShapeDtypeStruct((NC, NPAD, 16), jnp.float32),
    ),
    mesh=_mesh,
    scratch_types=[
        pltpu.VMEM((KSTAGE, CB), jnp.int32),
        pltpu.VMEM((KSTAGE, CB), jnp.int32),
        pltpu.VMEM((CB, EMBED), jnp.float32),
        pltpu.VMEM((CB, EMBED), jnp.float32),
        pltpu.VMEM((CB, EMBED), jnp.float32),
        pltpu.VMEM((CB, 16), jnp.float32),
        pltpu.VMEM_SHARED((NPAD, EMBED), jnp.float32),
        pltpu.VMEM_SHARED((NPAD, 16), jnp.float32),
        pltpu.SemaphoreType.DMA,
    ],
    compiler_params=pltpu.CompilerParams(use_tc_tiling_on_sc=False, needs_layout_passes=False),
)
def _sc_attention(q_hbm, k_hbm, v_hbm, r2d_hbm, c2d_hbm, z128_hbm, z8_hbm,
                  agg_out, seg_out, r_i, c_i, qb, kb, vb, exb,
                  agg_acc, seg_acc, sem):
    cid = lax.axis_index("c")
    sid = lax.axis_index("s")
    wid = cid * NS + sid
    pltpu.sync_copy(z128_hbm.at[pl.ds(sid * RPS, RPS)],
                    agg_acc.at[pl.ds(sid * RPS, RPS)])
    pltpu.sync_copy(z8_hbm.at[pl.ds(sid * RPS, RPS)],
                    seg_acc.at[pl.ds(sid * RPS, RPS)])
    plsc.subcore_barrier()

    def stage_body(s, carry0):
        pltpu.sync_copy(r2d_hbm.at[wid, pl.ds(s * KSTAGE, KSTAGE)], r_i)
        pltpu.sync_copy(c2d_hbm.at[wid, pl.ds(s * KSTAGE, KSTAGE)], c_i)
        pltpu.async_copy(q_hbm.at[c_i.at[0]], qb, sem)
        pltpu.async_copy(k_hbm.at[r_i.at[0]], kb, sem)
        pltpu.async_copy(v_hbm.at[r_i.at[0]], vb, sem)

        def chunk_body(jj, carry):
            cv = c_i.at[jj]
            pltpu.make_async_copy(q_hbm.at[pl.ds(0, CB)], qb, sem).wait()
            pltpu.make_async_copy(q_hbm.at[pl.ds(0, CB)], kb, sem).wait()
            pltpu.make_async_copy(q_hbm.at[pl.ds(0, CB)], vb, sem).wait()

            iota16 = lax.iota(jnp.int32, 16)

            @plsc.parallel_loop(0, CB, unroll=8)
            def edge_body(e):
                exrow = jnp.zeros((16,), jnp.float32)
                for h in range(NHEAD):
                    qv = qb[e, pl.ds(h * HDIM, HDIM)]
                    kv = kb[e, pl.ds(h * HDIM, HDIM)]
                    s = lax.reduce_sum(qv * kv, (0,)) * (1.0 / math.sqrt(HDIM))
                    exv = jnp.exp(lax.broadcast_in_dim(s, (16,), ()))
                    vb[e, pl.ds(h * HDIM, HDIM)] = (
                        vb[e, pl.ds(h * HDIM, HDIM)] * exv)
                    exrow = lax.select(iota16 == h, exv, exrow)
                plsc.store_scatter(
                    exb, [lax.broadcast_in_dim(e, (16,), ()), iota16], exrow)

            @pl.when(jj + 1 <= KSTAGE - 1)
            def _():
                pltpu.async_copy(q_hbm.at[c_i.at[jj + 1]], qb, sem)
                pltpu.async_copy(k_hbm.at[r_i.at[jj + 1]], kb, sem)

            pltpu.sync_copy(vb, agg_acc.at[cv], add=True)
            pltpu.sync_copy(exb, seg_acc.at[cv], add=True)

            @pl.when(jj + 1 <= KSTAGE - 1)
            def _():
                pltpu.async_copy(v_hbm.at[r_i.at[jj + 1]], vb, sem)
            return carry

        lax.fori_loop(0, KSTAGE, chunk_body, 0)
        return carry0

    lax.fori_loop(0, NSTAGE, stage_body, 0)
    plsc.subcore_barrier()
    pltpu.sync_copy(agg_acc.at[pl.ds(sid * RPS, RPS)],
                    agg_out.at[cid, pl.ds(sid * RPS, RPS)])
    pltpu.sync_copy(seg_acc.at[pl.ds(sid * RPS, RPS)],
                    seg_out.at[cid, pl.ds(sid * RPS, RPS)])


@functools.partial(
    pl.kernel,
    out_type=jax.ShapeDtypeStruct((NC, NPAD, EMBED), jnp.float32),
    mesh=_mesh,
    scratch_types=[
        pltpu.VMEM((NCHUNK, CB), jnp.int32),
        pltpu.VMEM((CB, EMBED), jnp.float32),
        pltpu.VMEM((CB, EMBED), jnp.float32),
        pltpu.VMEM_SHARED((NPAD, EMBED), jnp.float32),
        pltpu.SemaphoreType.DMA,
        pltpu.SemaphoreType.DMA,
    ],
    compiler_params=pltpu.CompilerParams(use_tc_tiling_on_sc=False, needs_layout_passes=False),
)
def _sc_edge_agg(na_hbm, r2d_hbm, z128_hbm, out_hbm, r_i, nb0, nb1, acc,
                 sem0, sem1):
    cid = lax.axis_index("c")
    sid = lax.axis_index("s")
    wid = cid * NS + sid
    pltpu.sync_copy(z128_hbm.at[pl.ds(sid * RPS, RPS)],
                    acc.at[pl.ds(sid * RPS, RPS)])
    pltpu.sync_copy(r2d_hbm.at[wid], r_i)
    plsc.subcore_barrier()

    nbufs = (nb0, nb1)
    sems = (sem0, sem1)
    base0 = wid * PER_W

    pltpu.async_copy(na_hbm.at[pl.ds(base0, CB)], nb0, sem0)
    pltpu.async_copy(na_hbm.at[pl.ds(base0 + CB, CB)], nb1, sem1)

    def pair_body(j2, carry):
        for b in range(2):
            t = j2 * 2 + b
            pltpu.make_async_copy(na_hbm.at[pl.ds(0, CB)], nbufs[b],
                                  sems[b]).wait()
            pltpu.sync_copy(nbufs[b], acc.at[r_i.at[t]], add=True)

            @pl.when(t + 2 <= NCHUNK - 1)
            def _():
                pltpu.async_copy(na_hbm.at[pl.ds(base0 + (t + 2) * CB, CB)],
                                 nbufs[b], sems[b])
        return carry

    lax.fori_loop(0, (NCHUNK - 1) // 2, pair_body, 0)
    pltpu.make_async_copy(na_hbm.at[pl.ds(0, CB)], nb0, sem0).wait()
    pltpu.sync_copy(nb0, acc.at[r_i.at[NCHUNK - 1]], add=True)
    plsc.subcore_barrier()
    pltpu.sync_copy(acc.at[pl.ds(sid * RPS, RPS)],
                    out_hbm.at[cid, pl.ds(sid * RPS, RPS)])


def _final_body(f_ref, agg_ref, seg_ref, eagg_ref, rep_ref, wo_ref, bo_ref,
                w1_ref, b1_ref, w2_ref, b2_ref, g_ref, bln_ref, o_ref):
    agg = agg_ref[0] + agg_ref[1]
    seg = seg_ref[0] + seg_ref[1]
    eagg = eagg_ref[0] + eagg_ref[1]
    den = jnp.dot(seg, rep_ref[...], preferred_element_type=jnp.float32) + 1e-16
    att = jnp.dot(agg / den, wo_ref[...], preferred_element_type=jnp.float32) \
        + bo_ref[...]
    g = g_ref[...]
    b = bln_ref[...]
    f1 = _ln(_gelu(f_ref[...] + att), g, b)
    f2 = _ln(_gelu(f1 + eagg), g, b)
    hdn = _gelu(jnp.dot(f2, w1_ref[...], preferred_element_type=jnp.float32)
                + b1_ref[...])
    ffn = jnp.dot(hdn, w2_ref[...], preferred_element_type=jnp.float32) + b2_ref[...]
    o_ref[...] = _ln(_gelu(f2 + ffn), g, b)


def _final(feats, agg2, seg2, eagg2, rep, Wo_t, bo, W1_t, b1, W2_t, b2, ln_g, ln_b):
    BN = 1000
    return pl.pallas_call(
        _final_body,
        grid=(N // BN,),
        in_specs=[
            pl.BlockSpec((BN, EMBED), lambda i: (i, 0)),
            pl.BlockSpec((NC, BN, EMBED), lambda i: (0, i, 0)),
            pl.BlockSpec((NC, BN, 16), lambda i: (0, i, 0)),
            pl.BlockSpec((NC, BN, EMBED), lambda i: (0, i, 0)),
            pl.BlockSpec((16, EMBED), lambda i: (0, 0)),
            pl.BlockSpec((EMBED, EMBED), lambda i: (0, 0)),
            pl.BlockSpec((1, EMBED), lambda i: (0, 0)),
            pl.BlockSpec((EMBED, DHID), lambda i: (0, 0)),
            pl.BlockSpec((1, DHID), lambda i: (0, 0)),
            pl.BlockSpec((DHID, EMBED), lambda i: (0, 0)),
            pl.BlockSpec((1, EMBED), lambda i: (0, 0)),
            pl.BlockSpec((1, EMBED), lambda i: (0, 0)),
            pl.BlockSpec((1, EMBED), lambda i: (0, 0)),
        ],
        out_specs=pl.BlockSpec((BN, EMBED), lambda i: (i, 0)),
        out_shape=jax.ShapeDtypeStruct((N, EMBED), jnp.float32),
    )(feats, agg2, seg2, eagg2, rep, Wo_t, bo, W1_t, b1, W2_t, b2, ln_g, ln_b)


def kernel(feats, edge_index, edge_attr, Wq, bq, Wk, bk, Wv, bv, Wo, bo,
           We, be, W1, b1, W2, b2, ln_g, ln_b):
    r2d = edge_index[:, 0].reshape(NW, NCHUNK, CB)
    c2d = edge_index[:, 1].reshape(NW, NCHUNK, CB)

    Wqkv_t = jnp.concatenate([Wq.T, Wk.T, Wv.T], axis=1)
    bqkv = jnp.concatenate([bq, bk, bv])[None, :]
    q, k, v = _qkv(feats, Wqkv_t, bqkv)

    z128 = jnp.zeros((NPAD, EMBED), jnp.float32)
    z8 = jnp.zeros((NPAD, 16), jnp.float32)

    agg2, seg2 = _sc_attention(q, k, v, r2d, c2d, z128, z8)

    node_attr = _edge_mlp(edge_attr, We.T, be[None, :])
    eagg2 = _sc_edge_agg(node_attr, r2d, z128)

    rep = jnp.concatenate(
        [jnp.repeat(jnp.eye(NHEAD, dtype=jnp.float32), HDIM, axis=1),
         jnp.zeros((16 - NHEAD, EMBED), jnp.float32)], axis=0)

    out = _final(feats, agg2, seg2, eagg2, rep, Wo.T, bo[None, :],
                 W1.T, b1[None, :], W2.T, b2[None, :], ln_g[None, :], ln_b[None, :])
    return (out, edge_index, edge_attr)
